# Initial kernel scaffold; baseline (speedup 1.0000x reference)
import jax
import jax.numpy as jnp
from jax.experimental import pallas as pl


def kernel(x_head, x_tail, edge_index_ht, edge_index_th, edge_label_index, Wsh0, Wnth0, bh0, Wst0, Wnht0, bt0, Wsh1, Wnth1, bh1, Wst1, Wnht1, bt1, lin1_W, lin1_b, lin2_W, lin2_b):
    raise NotImplementedError("write your pallas kernel here")


# speedup vs baseline: 3.7187x; 3.7187x over previous
import functools

import jax
import jax.numpy as jnp
from jax import lax
from jax.experimental import pallas as pl
from jax.experimental.pallas import tpu as pltpu
from jax.experimental.pallas import tpu_sc as plsc

N_NODE = 10000
E = 160000
D = 256
DH = 128
B = 4096
NS = 16
EPT = E // NS
CH = 200
RPT = 624
REM_BASE = RPT * NS
REM = N_NODE - REM_BASE
BPT = B // NS
DCH = 8

_sc_mesh = functools.partial(
    plsc.VectorSubcoreMesh, core_axis_name="c", subcore_axis_name="s")


def _zero_acc(zrows, acc, s):
    pltpu.sync_copy(zrows.at[pl.ds(0, RPT)], acc.at[pl.ds(s * RPT, RPT)])

    @pl.when(s == NS - 1)
    def _():
        pltpu.sync_copy(zrows.at[pl.ds(0, REM)], acc.at[pl.ds(REM_BASE, REM)])


def _flush(acc, out, s):
    pltpu.sync_copy(acc.at[pl.ds(s * RPT, RPT)], out.at[pl.ds(s * RPT, RPT)])

    @pl.when(s == NS - 1)
    def _():
        pltpu.sync_copy(acc.at[pl.ds(REM_BASE, REM)],
                        out.at[pl.ds(REM_BASE, REM)])


@functools.partial(
    pl.kernel,
    out_type=[
        jax.ShapeDtypeStruct((N_NODE, DH), jnp.float32),
        jax.ShapeDtypeStruct((N_NODE, DH), jnp.float32),
    ],
    mesh=_sc_mesh(),
    scratch_types=[
        pltpu.VMEM((CH,), jnp.int32),
        pltpu.VMEM((CH,), jnp.int32),
        pltpu.VMEM((CH, DH), jnp.float32),
        pltpu.VMEM_SHARED((N_NODE, DH), jnp.float32),
    ],
)
def _sc_agg(h_L, h_R, src, dst, zrows,
            S_L, S_R,
            sidx, didx, rows, acc):
    c = lax.axis_index("c")
    s = lax.axis_index("s")
    ebase = s * EPT

    _zero_acc(zrows, acc, s)
    plsc.subcore_barrier()

    @pl.when(c == 0)
    def _():
        def chunk(i, carry):
            base = ebase + i * CH
            pltpu.sync_copy(src.at[pl.ds(base, CH)], sidx)
            pltpu.sync_copy(dst.at[pl.ds(base, CH)], didx)
            pltpu.sync_copy(h_L.at[sidx], rows)
            pltpu.sync_copy(rows, acc.at[didx], add=True)
            return carry

        lax.fori_loop(0, EPT // CH, chunk, 0)

    @pl.when(c == 1)
    def _():
        def chunk(i, carry):
            base = ebase + i * CH
            pltpu.sync_copy(src.at[pl.ds(base, CH)], sidx)
            pltpu.sync_copy(dst.at[pl.ds(base, CH)], didx)
            pltpu.sync_copy(h_R.at[sidx], rows)
            pltpu.sync_copy(rows, acc.at[didx], add=True)
            return carry

        lax.fori_loop(0, EPT // CH, chunk, 0)

    plsc.subcore_barrier()

    @pl.when(c == 0)
    def _():
        _flush(acc, S_L, s)

    @pl.when(c == 1)
    def _():
        _flush(acc, S_R, s)


CH_C = 1000


def _zero_1d(zbuf, acc, s):
    pltpu.sync_copy(zbuf.at[pl.ds(0, RPT)], acc.at[pl.ds(s * RPT, RPT)])

    @pl.when(s == NS - 1)
    def _():
        pltpu.sync_copy(zbuf.at[pl.ds(0, REM)], acc.at[pl.ds(REM_BASE, REM)])


def _flush_1d(acc, cbuf, out, s):
    pltpu.sync_copy(acc.at[pl.ds(s * RPT, RPT)], cbuf.at[pl.ds(0, RPT)])
    pltpu.sync_copy(cbuf.at[pl.ds(0, RPT)], out.at[pl.ds(s * RPT, RPT)])

    @pl.when(s == NS - 1)
    def _():
        pltpu.sync_copy(acc.at[pl.ds(REM_BASE, REM)], cbuf.at[pl.ds(0, REM)])
        pltpu.sync_copy(cbuf.at[pl.ds(0, REM)], out.at[pl.ds(REM_BASE, REM)])


@functools.partial(
    pl.kernel,
    out_type=[
        jax.ShapeDtypeStruct((N_NODE,), jnp.float32),
        jax.ShapeDtypeStruct((N_NODE,), jnp.float32),
    ],
    mesh=_sc_mesh(),
    scratch_types=[
        pltpu.VMEM((CH_C,), jnp.int32),
        pltpu.VMEM((CH_C,), jnp.float32),
        pltpu.VMEM((RPT,), jnp.float32),
        pltpu.VMEM_SHARED((N_NODE,), jnp.float32),
    ],
)
def _sc_cnt(dst_ht, dst_th, zrows1, ones1,
            cnt_t, cnt_h,
            didx, ones_v, cbuf, acc_cnt):
    c = lax.axis_index("c")
    s = lax.axis_index("s")
    ebase = s * EPT

    pltpu.sync_copy(zrows1, cbuf)
    _zero_1d(cbuf, acc_cnt, s)
    pltpu.sync_copy(ones1, ones_v)
    plsc.subcore_barrier()

    def count_chunks(dst):
        def chunk(i, carry):
            base = ebase + i * CH_C
            pltpu.sync_copy(dst.at[pl.ds(base, CH_C)], didx)
            pltpu.sync_copy(ones_v, acc_cnt.at[didx], add=True)
            return carry

        lax.fori_loop(0, EPT // CH_C, chunk, 0)

    @pl.when(c == 0)
    def _():
        count_chunks(dst_ht)

    @pl.when(c == 1)
    def _():
        count_chunks(dst_th)

    plsc.subcore_barrier()

    @pl.when(c == 0)
    def _():
        _flush_1d(acc_cnt, cbuf, cnt_t, s)

    @pl.when(c == 1)
    def _():
        _flush_1d(acc_cnt, cbuf, cnt_h, s)


@functools.partial(
    pl.kernel,
    out_type=[
        jax.ShapeDtypeStruct((B, D), jnp.float32),
        jax.ShapeDtypeStruct((B, D), jnp.float32),
    ],
    mesh=_sc_mesh(),
    scratch_types=[
        pltpu.VMEM((BPT,), jnp.int32),
        pltpu.VMEM((DCH, D), jnp.float32),
    ],
)
def _sc_dec_gather(h_tail, eli0, eli1, z_src, z_dst, idx_v, rows):
    c = lax.axis_index("c")
    s = lax.axis_index("s")
    base = s * BPT

    def gather_to(eli, z_out):
        pltpu.sync_copy(eli.at[pl.ds(base, BPT)], idx_v)

        def chunk(k, carry):
            pltpu.sync_copy(h_tail.at[idx_v.at[pl.ds(k * DCH, DCH)]], rows)
            pltpu.sync_copy(rows, z_out.at[pl.ds(base + k * DCH, DCH)])
            return carry

        lax.fori_loop(0, BPT // DCH, chunk, 0)

    @pl.when(c == 0)
    def _():
        gather_to(eli0, z_src)

    @pl.when(c == 1)
    def _():
        gather_to(eli1, z_dst)


_TCR = 1000


def _tc_layer_body(xh, xt, StL, StR, ShL, ShR, cnt_t, cnt_h,
                   Wsh, Wnth, bh, Wst, Wnht, bt,
                   oh_L, oh_R, ot_L, ot_R):
    f32 = jnp.float32
    rt = 1.0 / jnp.maximum(cnt_t[...], 1.0)
    rh = 1.0 / jnp.maximum(cnt_h[...], 1.0)
    nh = jnp.dot(xh[...], Wsh[...], preferred_element_type=f32)
    nh = nh + jnp.dot(ShL[...] * rh, Wnth[:DH, :], preferred_element_type=f32)
    nh = nh + jnp.dot(ShR[...] * rh, Wnth[DH:, :], preferred_element_type=f32)
    nh = jnp.maximum(nh + bh[...], 0.0)
    oh_L[...] = nh[:, :DH]
    oh_R[...] = nh[:, DH:]
    nt = jnp.dot(xt[...], Wst[...], preferred_element_type=f32)
    nt = nt + jnp.dot(StL[...] * rt, Wnht[:DH, :], preferred_element_type=f32)
    nt = nt + jnp.dot(StR[...] * rt, Wnht[DH:, :], preferred_element_type=f32)
    nt = jnp.maximum(nt + bt[...], 0.0)
    ot_L[...] = nt[:, :DH]
    ot_R[...] = nt[:, DH:]


def _row_spec(w):
    return pl.BlockSpec((_TCR, w), lambda i: (i, 0))


def _full_spec(shape):
    return pl.BlockSpec(shape, lambda i: (0,) * len(shape))


def _tc_layer(xh, xt, StL, StR, ShL, ShR, cnt_t, cnt_h,
              Wsh, Wnth, bh, Wst, Wnht, bt):
    grid = (N_NODE // _TCR,)
    half = jax.ShapeDtypeStruct((N_NODE, DH), jnp.float32)
    return pl.pallas_call(
        _tc_layer_body,
        grid=grid,
        in_specs=[
            _row_spec(D), _row_spec(D),
            _row_spec(DH), _row_spec(DH), _row_spec(DH), _row_spec(DH),
            _row_spec(1), _row_spec(1),
            _full_spec((D, D)), _full_spec((D, D)), _full_spec((1, D)),
            _full_spec((D, D)), _full_spec((D, D)), _full_spec((1, D)),
        ],
        out_specs=[_row_spec(DH)] * 4,
        out_shape=[half] * 4,
        compiler_params=pltpu.CompilerParams(
            dimension_semantics=("parallel",)),
    )(xh, xt, StL, StR, ShL, ShR, cnt_t, cnt_h,
      Wsh, Wnth, bh, Wst, Wnht, bt)


def _tc_tail_body(htL, htR, StL, StR, cnt_t, Wst, Wnht, bt, out):
    f32 = jnp.float32
    rt = 1.0 / jnp.maximum(cnt_t[...], 1.0)
    nt = jnp.dot(htL[...], Wst[:DH, :], preferred_element_type=f32)
    nt = nt + jnp.dot(htR[...], Wst[DH:, :], preferred_element_type=f32)
    nt = nt + jnp.dot(StL[...] * rt, Wnht[:DH, :], preferred_element_type=f32)
    nt = nt + jnp.dot(StR[...] * rt, Wnht[DH:, :], preferred_element_type=f32)
    out[...] = jnp.maximum(nt + bt[...], 0.0)


def _tc_tail(htL, htR, StL, StR, cnt_t, Wst, Wnht, bt):
    grid = (N_NODE // _TCR,)
    return pl.pallas_call(
        _tc_tail_body,
        grid=grid,
        in_specs=[
            _row_spec(DH), _row_spec(DH), _row_spec(DH), _row_spec(DH),
            _row_spec(1),
            _full_spec((D, D)), _full_spec((D, D)), _full_spec((1, D)),
        ],
        out_specs=_row_spec(D),
        out_shape=jax.ShapeDtypeStruct((N_NODE, D), jnp.float32),
        compiler_params=pltpu.CompilerParams(
            dimension_semantics=("parallel",)),
    )(htL, htR, StL, StR, cnt_t, Wst, Wnht, bt)


def _tc_dec_body(zs, zd, W1a, W1b, b1, W2, b2, out):
    f32 = jnp.float32
    x = jnp.dot(zs[...], W1a[...], preferred_element_type=f32)
    x = x + jnp.dot(zd[...], W1b[...], preferred_element_type=f32)
    x = jnp.maximum(x + b1[...], 0.0)
    out[...] = jnp.dot(x, W2[...], preferred_element_type=f32) + b2[...]


def _tc_dec(z_src, z_dst, W1a, W1b, b1, W2, b2):
    return pl.pallas_call(
        _tc_dec_body,
        out_shape=jax.ShapeDtypeStruct((B, 1), jnp.float32),
    )(z_src, z_dst, W1a, W1b, b1, W2, b2)


def kernel(x_head, x_tail, edge_index_ht, edge_index_th, edge_label_index,
           Wsh0, Wnth0, bh0, Wst0, Wnht0, bt0,
           Wsh1, Wnth1, bh1, Wst1, Wnht1, bt1,
           lin1_W, lin1_b, lin2_W, lin2_b):
    i32 = jnp.int32
    src_ht = edge_index_ht[0].astype(i32)
    dst_ht = edge_index_ht[1].astype(i32)
    src_th = edge_index_th[0].astype(i32)
    dst_th = edge_index_th[1].astype(i32)
    eli0 = edge_label_index[0].astype(i32)
    eli1 = edge_label_index[1].astype(i32)

    xh_L, xh_R = x_head[:, :DH], x_head[:, DH:]
    xt_L, xt_R = x_tail[:, :DH], x_tail[:, DH:]

    zrows = jnp.zeros((RPT, DH), jnp.float32)
    zrows1 = jnp.zeros((RPT,), jnp.float32)
    ones1 = jnp.ones((CH_C,), jnp.float32)

    cnt_t1, cnt_h1 = _sc_cnt(dst_ht, dst_th, zrows1, ones1)
    cnt_t = cnt_t1.reshape(N_NODE, 1)
    cnt_h = cnt_h1.reshape(N_NODE, 1)
    StL, StR = _sc_agg(xh_L, xh_R, src_ht, dst_ht, zrows)
    ShL, ShR = _sc_agg(xt_L, xt_R, src_th, dst_th, zrows)

    h1hL, h1hR, h1tL, h1tR = _tc_layer(
        x_head, x_tail, StL, StR, ShL, ShR, cnt_t, cnt_h,
        Wsh0, Wnth0, bh0.reshape(1, D), Wst0, Wnht0, bt0.reshape(1, D))

    S1tL, S1tR = _sc_agg(h1hL, h1hR, src_ht, dst_ht, zrows)

    h2_tail = _tc_tail(h1tL, h1tR, S1tL, S1tR, cnt_t,
                       Wst1, Wnht1, bt1.reshape(1, D))

    z_src, z_dst = _sc_dec_gather(h2_tail, eli0, eli1)

    out = _tc_dec(z_src, z_dst, lin1_W[:D], lin1_W[D:],
                  lin1_b.reshape(1, D), lin2_W, lin2_b.reshape(1, 1))
    return out.reshape(-1)


# speedup vs baseline: 3.8349x; 1.0312x over previous
import functools

import jax
import jax.numpy as jnp
from jax import lax
from jax.experimental import pallas as pl
from jax.experimental.pallas import tpu as pltpu
from jax.experimental.pallas import tpu_sc as plsc

N_NODE = 10000
E = 160000
D = 256
DH = 128
B = 4096
NS = 16
EPT = E // NS
CH = 40
RPT = 624
REM_BASE = RPT * NS
REM = N_NODE - REM_BASE
BPT = B // NS
DCH = 8

_sc_mesh = functools.partial(
    plsc.VectorSubcoreMesh, core_axis_name="c", subcore_axis_name="s")


def _zero_acc(zrows, acc, s):
    pltpu.sync_copy(zrows.at[pl.ds(0, RPT)], acc.at[pl.ds(s * RPT, RPT)])

    @pl.when(s == NS - 1)
    def _():
        pltpu.sync_copy(zrows.at[pl.ds(0, REM)], acc.at[pl.ds(REM_BASE, REM)])


def _flush(acc, out, s):
    pltpu.sync_copy(acc.at[pl.ds(s * RPT, RPT)], out.at[pl.ds(s * RPT, RPT)])

    @pl.when(s == NS - 1)
    def _():
        pltpu.sync_copy(acc.at[pl.ds(REM_BASE, REM)],
                        out.at[pl.ds(REM_BASE, REM)])


NPAIR = EPT // (2 * CH)


def _agg_pipelined(tbl, src, dst, acc, ebase,
                   sidx0, didx0, rows0, sidx1, didx1, rows1,
                   sem_i0, sem_i1, sem_g0, sem_g1, sem_s):
    def idx_start(k, si, di, sem):
        base = ebase + k * CH
        pltpu.async_copy(src.at[pl.ds(base, CH)], si, sem)
        pltpu.async_copy(dst.at[pl.ds(base, CH)], di, sem)

    def idx_wait(si, di, sem):
        pltpu.make_async_copy(src.at[pl.ds(ebase, CH)], si, sem).wait()
        pltpu.make_async_copy(dst.at[pl.ds(ebase, CH)], di, sem).wait()

    def gather_start(si, rows, sem):
        pltpu.async_copy(tbl.at[si], rows, sem)

    def gather_wait(si, rows, sem):
        pltpu.make_async_copy(tbl.at[si], rows, sem).wait()

    idx_start(0, sidx0, didx0, sem_i0)
    idx_start(1, sidx1, didx1, sem_i1)
    idx_wait(sidx0, didx0, sem_i0)
    gather_start(sidx0, rows0, sem_g0)

    def pair(j, carry):
        a = 2 * j
        idx_wait(sidx1, didx1, sem_i1)
        gather_start(sidx1, rows1, sem_g1)
        gather_wait(sidx0, rows0, sem_g0)

        @pl.when(j < NPAIR - 1)
        def _():
            idx_start(a + 2, sidx0, didx0, sem_i0)
            pltpu.async_copy(rows0, acc.at[didx0], sem_s, add=True)

        @pl.when(j == NPAIR - 1)
        def _():
            pltpu.sync_copy(rows0, acc.at[didx0], add=True)

        gather_wait(sidx1, rows1, sem_g1)

        @pl.when(j < NPAIR - 1)
        def _():
            idx_start(a + 3, sidx1, didx1, sem_i1)

        pltpu.sync_copy(rows1, acc.at[didx1], add=True)

        @pl.when(j < NPAIR - 1)
        def _():
            pltpu.make_async_copy(rows0, acc.at[didx0], sem_s).wait()
            idx_wait(sidx0, didx0, sem_i0)
            gather_start(sidx0, rows0, sem_g0)

        return carry

    lax.fori_loop(0, NPAIR, pair, 0)


@functools.partial(
    pl.kernel,
    out_type=[
        jax.ShapeDtypeStruct((N_NODE, DH), jnp.float32),
        jax.ShapeDtypeStruct((N_NODE, DH), jnp.float32),
    ],
    mesh=_sc_mesh(),
    scratch_types=[
        pltpu.VMEM((CH,), jnp.int32),
        pltpu.VMEM((CH,), jnp.int32),
        pltpu.VMEM((CH, DH), jnp.float32),
        pltpu.VMEM((CH,), jnp.int32),
        pltpu.VMEM((CH,), jnp.int32),
        pltpu.VMEM((CH, DH), jnp.float32),
        pltpu.VMEM_SHARED((N_NODE, DH), jnp.float32),
        pltpu.SemaphoreType.DMA,
        pltpu.SemaphoreType.DMA,
        pltpu.SemaphoreType.DMA,
        pltpu.SemaphoreType.DMA,
        pltpu.SemaphoreType.DMA,
    ],
)
def _sc_agg(h_L, h_R, src, dst, zrows,
            S_L, S_R,
            sidx0, didx0, rows0, sidx1, didx1, rows1, acc,
            sem_i0, sem_i1, sem_g0, sem_g1, sem_s):
    c = lax.axis_index("c")
    s = lax.axis_index("s")
    ebase = s * EPT

    _zero_acc(zrows, acc, s)
    plsc.subcore_barrier()

    @pl.when(c == 0)
    def _():
        _agg_pipelined(h_L, src, dst, acc, ebase,
                       sidx0, didx0, rows0, sidx1, didx1, rows1,
                       sem_i0, sem_i1, sem_g0, sem_g1, sem_s)

    @pl.when(c == 1)
    def _():
        _agg_pipelined(h_R, src, dst, acc, ebase,
                       sidx0, didx0, rows0, sidx1, didx1, rows1,
                       sem_i0, sem_i1, sem_g0, sem_g1, sem_s)

    plsc.subcore_barrier()

    @pl.when(c == 0)
    def _():
        _flush(acc, S_L, s)

    @pl.when(c == 1)
    def _():
        _flush(acc, S_R, s)


CH_C = 1000


def _zero_1d(zbuf, acc, s):
    pltpu.sync_copy(zbuf.at[pl.ds(0, RPT)], acc.at[pl.ds(s * RPT, RPT)])

    @pl.when(s == NS - 1)
    def _():
        pltpu.sync_copy(zbuf.at[pl.ds(0, REM)], acc.at[pl.ds(REM_BASE, REM)])


def _flush_1d(acc, cbuf, out, s):
    pltpu.sync_copy(acc.at[pl.ds(s * RPT, RPT)], cbuf.at[pl.ds(0, RPT)])
    pltpu.sync_copy(cbuf.at[pl.ds(0, RPT)], out.at[pl.ds(s * RPT, RPT)])

    @pl.when(s == NS - 1)
    def _():
        pltpu.sync_copy(acc.at[pl.ds(REM_BASE, REM)], cbuf.at[pl.ds(0, REM)])
        pltpu.sync_copy(cbuf.at[pl.ds(0, REM)], out.at[pl.ds(REM_BASE, REM)])


@functools.partial(
    pl.kernel,
    out_type=[
        jax.ShapeDtypeStruct((N_NODE,), jnp.float32),
        jax.ShapeDtypeStruct((N_NODE,), jnp.float32),
    ],
    mesh=_sc_mesh(),
    scratch_types=[
        pltpu.VMEM((CH_C,), jnp.int32),
        pltpu.VMEM((CH_C,), jnp.float32),
        pltpu.VMEM((RPT,), jnp.float32),
        pltpu.VMEM_SHARED((N_NODE,), jnp.float32),
    ],
)
def _sc_cnt(dst_ht, dst_th, zrows1, ones1,
            cnt_t, cnt_h,
            didx, ones_v, cbuf, acc_cnt):
    c = lax.axis_index("c")
    s = lax.axis_index("s")
    ebase = s * EPT

    pltpu.sync_copy(zrows1, cbuf)
    _zero_1d(cbuf, acc_cnt, s)
    pltpu.sync_copy(ones1, ones_v)
    plsc.subcore_barrier()

    def count_chunks(dst):
        def chunk(i, carry):
            base = ebase + i * CH_C
            pltpu.sync_copy(dst.at[pl.ds(base, CH_C)], didx)
            pltpu.sync_copy(ones_v, acc_cnt.at[didx], add=True)
            return carry

        lax.fori_loop(0, EPT // CH_C, chunk, 0)

    @pl.when(c == 0)
    def _():
        count_chunks(dst_ht)

    @pl.when(c == 1)
    def _():
        count_chunks(dst_th)

    plsc.subcore_barrier()

    @pl.when(c == 0)
    def _():
        _flush_1d(acc_cnt, cbuf, cnt_t, s)

    @pl.when(c == 1)
    def _():
        _flush_1d(acc_cnt, cbuf, cnt_h, s)


@functools.partial(
    pl.kernel,
    out_type=[
        jax.ShapeDtypeStruct((B, D), jnp.float32),
        jax.ShapeDtypeStruct((B, D), jnp.float32),
    ],
    mesh=_sc_mesh(),
    scratch_types=[
        pltpu.VMEM((BPT,), jnp.int32),
        pltpu.VMEM((DCH, D), jnp.float32),
    ],
)
def _sc_dec_gather(h_tail, eli0, eli1, z_src, z_dst, idx_v, rows):
    c = lax.axis_index("c")
    s = lax.axis_index("s")
    base = s * BPT

    def gather_to(eli, z_out):
        pltpu.sync_copy(eli.at[pl.ds(base, BPT)], idx_v)

        def chunk(k, carry):
            pltpu.sync_copy(h_tail.at[idx_v.at[pl.ds(k * DCH, DCH)]], rows)
            pltpu.sync_copy(rows, z_out.at[pl.ds(base + k * DCH, DCH)])
            return carry

        lax.fori_loop(0, BPT // DCH, chunk, 0)

    @pl.when(c == 0)
    def _():
        gather_to(eli0, z_src)

    @pl.when(c == 1)
    def _():
        gather_to(eli1, z_dst)


_TCR = 1000


def _tc_layer_body(xh, xt, StL, StR, ShL, ShR, cnt_t, cnt_h,
                   Wsh, Wnth, bh, Wst, Wnht, bt,
                   oh_L, oh_R, ot_L, ot_R):
    f32 = jnp.float32
    rt = 1.0 / jnp.maximum(cnt_t[...], 1.0)
    rh = 1.0 / jnp.maximum(cnt_h[...], 1.0)
    nh = jnp.dot(xh[...], Wsh[...], preferred_element_type=f32)
    nh = nh + jnp.dot(ShL[...] * rh, Wnth[:DH, :], preferred_element_type=f32)
    nh = nh + jnp.dot(ShR[...] * rh, Wnth[DH:, :], preferred_element_type=f32)
    nh = jnp.maximum(nh + bh[...], 0.0)
    oh_L[...] = nh[:, :DH]
    oh_R[...] = nh[:, DH:]
    nt = jnp.dot(xt[...], Wst[...], preferred_element_type=f32)
    nt = nt + jnp.dot(StL[...] * rt, Wnht[:DH, :], preferred_element_type=f32)
    nt = nt + jnp.dot(StR[...] * rt, Wnht[DH:, :], preferred_element_type=f32)
    nt = jnp.maximum(nt + bt[...], 0.0)
    ot_L[...] = nt[:, :DH]
    ot_R[...] = nt[:, DH:]


def _row_spec(w):
    return pl.BlockSpec((_TCR, w), lambda i: (i, 0))


def _full_spec(shape):
    return pl.BlockSpec(shape, lambda i: (0,) * len(shape))


def _tc_layer(xh, xt, StL, StR, ShL, ShR, cnt_t, cnt_h,
              Wsh, Wnth, bh, Wst, Wnht, bt):
    grid = (N_NODE // _TCR,)
    half = jax.ShapeDtypeStruct((N_NODE, DH), jnp.float32)
    return pl.pallas_call(
        _tc_layer_body,
        grid=grid,
        in_specs=[
            _row_spec(D), _row_spec(D),
            _row_spec(DH), _row_spec(DH), _row_spec(DH), _row_spec(DH),
            _row_spec(1), _row_spec(1),
            _full_spec((D, D)), _full_spec((D, D)), _full_spec((1, D)),
            _full_spec((D, D)), _full_spec((D, D)), _full_spec((1, D)),
        ],
        out_specs=[_row_spec(DH)] * 4,
        out_shape=[half] * 4,
        compiler_params=pltpu.CompilerParams(
            dimension_semantics=("parallel",)),
    )(xh, xt, StL, StR, ShL, ShR, cnt_t, cnt_h,
      Wsh, Wnth, bh, Wst, Wnht, bt)


def _tc_tail_body(htL, htR, StL, StR, cnt_t, Wst, Wnht, bt, out):
    f32 = jnp.float32
    rt = 1.0 / jnp.maximum(cnt_t[...], 1.0)
    nt = jnp.dot(htL[...], Wst[:DH, :], preferred_element_type=f32)
    nt = nt + jnp.dot(htR[...], Wst[DH:, :], preferred_element_type=f32)
    nt = nt + jnp.dot(StL[...] * rt, Wnht[:DH, :], preferred_element_type=f32)
    nt = nt + jnp.dot(StR[...] * rt, Wnht[DH:, :], preferred_element_type=f32)
    out[...] = jnp.maximum(nt + bt[...], 0.0)


def _tc_tail(htL, htR, StL, StR, cnt_t, Wst, Wnht, bt):
    grid = (N_NODE // _TCR,)
    return pl.pallas_call(
        _tc_tail_body,
        grid=grid,
        in_specs=[
            _row_spec(DH), _row_spec(DH), _row_spec(DH), _row_spec(DH),
            _row_spec(1),
            _full_spec((D, D)), _full_spec((D, D)), _full_spec((1, D)),
        ],
        out_specs=_row_spec(D),
        out_shape=jax.ShapeDtypeStruct((N_NODE, D), jnp.float32),
        compiler_params=pltpu.CompilerParams(
            dimension_semantics=("parallel",)),
    )(htL, htR, StL, StR, cnt_t, Wst, Wnht, bt)


def _tc_dec_body(zs, zd, W1a, W1b, b1, W2, b2, out):
    f32 = jnp.float32
    x = jnp.dot(zs[...], W1a[...], preferred_element_type=f32)
    x = x + jnp.dot(zd[...], W1b[...], preferred_element_type=f32)
    x = jnp.maximum(x + b1[...], 0.0)
    out[...] = jnp.dot(x, W2[...], preferred_element_type=f32) + b2[...]


def _tc_dec(z_src, z_dst, W1a, W1b, b1, W2, b2):
    return pl.pallas_call(
        _tc_dec_body,
        out_shape=jax.ShapeDtypeStruct((B, 1), jnp.float32),
    )(z_src, z_dst, W1a, W1b, b1, W2, b2)


def kernel(x_head, x_tail, edge_index_ht, edge_index_th, edge_label_index,
           Wsh0, Wnth0, bh0, Wst0, Wnht0, bt0,
           Wsh1, Wnth1, bh1, Wst1, Wnht1, bt1,
           lin1_W, lin1_b, lin2_W, lin2_b):
    i32 = jnp.int32
    src_ht = edge_index_ht[0].astype(i32)
    dst_ht = edge_index_ht[1].astype(i32)
    src_th = edge_index_th[0].astype(i32)
    dst_th = edge_index_th[1].astype(i32)
    eli0 = edge_label_index[0].astype(i32)
    eli1 = edge_label_index[1].astype(i32)

    xh_L, xh_R = x_head[:, :DH], x_head[:, DH:]
    xt_L, xt_R = x_tail[:, :DH], x_tail[:, DH:]

    zrows = jnp.zeros((RPT, DH), jnp.float32)
    zrows1 = jnp.zeros((RPT,), jnp.float32)
    ones1 = jnp.ones((CH_C,), jnp.float32)

    cnt_t1, cnt_h1 = _sc_cnt(dst_ht, dst_th, zrows1, ones1)
    cnt_t = cnt_t1.reshape(N_NODE, 1)
    cnt_h = cnt_h1.reshape(N_NODE, 1)
    StL, StR = _sc_agg(xh_L, xh_R, src_ht, dst_ht, zrows)
    ShL, ShR = _sc_agg(xt_L, xt_R, src_th, dst_th, zrows)

    h1hL, h1hR, h1tL, h1tR = _tc_layer(
        x_head, x_tail, StL, StR, ShL, ShR, cnt_t, cnt_h,
        Wsh0, Wnth0, bh0.reshape(1, D), Wst0, Wnht0, bt0.reshape(1, D))

    S1tL, S1tR = _sc_agg(h1hL, h1hR, src_ht, dst_ht, zrows)

    h2_tail = _tc_tail(h1tL, h1tR, S1tL, S1tR, cnt_t,
                       Wst1, Wnht1, bt1.reshape(1, D))

    z_src, z_dst = _sc_dec_gather(h2_tail, eli0, eli1)

    out = _tc_dec(z_src, z_dst, lin1_W[:D], lin1_W[D:],
                  lin1_b.reshape(1, D), lin2_W, lin2_b.reshape(1, 1))
    return out.reshape(-1)


# speedup vs baseline: 5.0160x; 1.3080x over previous
import functools

import jax
import jax.numpy as jnp
from jax import lax
from jax.experimental import pallas as pl
from jax.experimental.pallas import tpu as pltpu
from jax.experimental.pallas import tpu_sc as plsc

N_NODE = 10000
E = 160000
D = 256
DH = 128
B = 4096
NS = 16
EPT = E // NS
CH = 40
RPT = 624
REM_BASE = RPT * NS
REM = N_NODE - REM_BASE
BPT = B // NS
DCH = 8

_sc_mesh = functools.partial(
    plsc.VectorSubcoreMesh, core_axis_name="c", subcore_axis_name="s")


def _zero_acc(zrows, acc, s):
    pltpu.sync_copy(zrows.at[pl.ds(0, RPT)], acc.at[pl.ds(s * RPT, RPT)])

    @pl.when(s == NS - 1)
    def _():
        pltpu.sync_copy(zrows.at[pl.ds(0, REM)], acc.at[pl.ds(REM_BASE, REM)])


def _flush(acc, out, s):
    pltpu.sync_copy(acc.at[pl.ds(s * RPT, RPT)], out.at[pl.ds(s * RPT, RPT)])

    @pl.when(s == NS - 1)
    def _():
        pltpu.sync_copy(acc.at[pl.ds(REM_BASE, REM)],
                        out.at[pl.ds(REM_BASE, REM)])


NSLOT = 5
NCHUNK = EPT // CH
NGROUP = NCHUNK // NSLOT


def _agg_pipelined(tbl, src, dst, acc, ebase, slots):
    def idx_start(k, sl):
        base = ebase + k * CH
        pltpu.async_copy(src.at[pl.ds(base, CH)], sl["si"], sl["sem_i"])
        pltpu.async_copy(dst.at[pl.ds(base, CH)], sl["di"], sl["sem_i"])

    def idx_wait(sl):
        pltpu.make_async_copy(src.at[pl.ds(ebase, CH)], sl["si"],
                              sl["sem_i"]).wait()
        pltpu.make_async_copy(dst.at[pl.ds(ebase, CH)], sl["di"],
                              sl["sem_i"]).wait()

    def gather_start(sl):
        pltpu.async_copy(tbl.at[sl["si"]], sl["rows"], sl["sem_g"])

    def gather_wait(sl):
        pltpu.make_async_copy(tbl.at[sl["si"]], sl["rows"],
                              sl["sem_g"]).wait()

    def scat_start(sl):
        pltpu.async_copy(sl["rows"], acc.at[sl["di"]], sl["sem_s"], add=True)

    def scat_drain(sl):
        pltpu.make_async_copy(sl["rows"], acc.at[sl["di"]],
                              sl["sem_s"]).wait()

    idx_start(0, slots[0])
    idx_start(1, slots[1])

    def step(k, u):
        cur = slots[u]
        prev = slots[(u - 1) % NSLOT]
        prev2 = slots[(u - 2) % NSLOT]
        ahead2 = slots[(u + 2) % NSLOT]

        idx_wait(cur)
        gather_start(cur)

        @pl.when(k >= 1)
        def _():
            gather_wait(prev)
            scat_start(prev)

        @pl.when(k >= 2)
        def _():
            scat_drain(prev2)

        @pl.when(k + 2 < NCHUNK)
        def _():
            idx_start(k + 2, ahead2)

    def group(jj, carry):
        for u in range(NSLOT):
            step(jj * NSLOT + u, u)
        return carry

    lax.fori_loop(0, NGROUP, group, 0)

    last = slots[(NCHUNK - 1) % NSLOT]
    gather_wait(last)
    scat_start(last)
    scat_drain(slots[(NCHUNK - 2) % NSLOT])
    scat_drain(last)


@functools.partial(
    pl.kernel,
    out_type=[
        jax.ShapeDtypeStruct((N_NODE, DH), jnp.float32),
        jax.ShapeDtypeStruct((N_NODE, DH), jnp.float32),
    ],
    mesh=_sc_mesh(),
    scratch_types=(
        [pltpu.VMEM((CH,), jnp.int32)] * (2 * NSLOT)
        + [pltpu.VMEM((CH, DH), jnp.float32)] * NSLOT
        + [pltpu.VMEM_SHARED((N_NODE, DH), jnp.float32)]
        + [pltpu.SemaphoreType.DMA] * (3 * NSLOT)
    ),
)
def _sc_agg(h_L, h_R, src, dst, zrows, S_L, S_R, *scr):
    sis = scr[0:NSLOT]
    dis = scr[NSLOT:2 * NSLOT]
    rows = scr[2 * NSLOT:3 * NSLOT]
    acc = scr[3 * NSLOT]
    sem_i = scr[3 * NSLOT + 1:3 * NSLOT + 1 + NSLOT]
    sem_g = scr[3 * NSLOT + 1 + NSLOT:3 * NSLOT + 1 + 2 * NSLOT]
    sem_s = scr[3 * NSLOT + 1 + 2 * NSLOT:3 * NSLOT + 1 + 3 * NSLOT]
    slots = [dict(si=sis[u], di=dis[u], rows=rows[u],
                  sem_i=sem_i[u], sem_g=sem_g[u], sem_s=sem_s[u])
             for u in range(NSLOT)]

    c = lax.axis_index("c")
    s = lax.axis_index("s")
    ebase = s * EPT

    _zero_acc(zrows, acc, s)
    plsc.subcore_barrier()

    @pl.when(c == 0)
    def _():
        _agg_pipelined(h_L, src, dst, acc, ebase, slots)

    @pl.when(c == 1)
    def _():
        _agg_pipelined(h_R, src, dst, acc, ebase, slots)

    plsc.subcore_barrier()

    @pl.when(c == 0)
    def _():
        _flush(acc, S_L, s)

    @pl.when(c == 1)
    def _():
        _flush(acc, S_R, s)


CH_C = 1000


def _zero_1d(zbuf, acc, s):
    pltpu.sync_copy(zbuf.at[pl.ds(0, RPT)], acc.at[pl.ds(s * RPT, RPT)])

    @pl.when(s == NS - 1)
    def _():
        pltpu.sync_copy(zbuf.at[pl.ds(0, REM)], acc.at[pl.ds(REM_BASE, REM)])


def _flush_1d(acc, cbuf, out, s):
    pltpu.sync_copy(acc.at[pl.ds(s * RPT, RPT)], cbuf.at[pl.ds(0, RPT)])
    pltpu.sync_copy(cbuf.at[pl.ds(0, RPT)], out.at[pl.ds(s * RPT, RPT)])

    @pl.when(s == NS - 1)
    def _():
        pltpu.sync_copy(acc.at[pl.ds(REM_BASE, REM)], cbuf.at[pl.ds(0, REM)])
        pltpu.sync_copy(cbuf.at[pl.ds(0, REM)], out.at[pl.ds(REM_BASE, REM)])


@functools.partial(
    pl.kernel,
    out_type=[
        jax.ShapeDtypeStruct((N_NODE,), jnp.float32),
        jax.ShapeDtypeStruct((N_NODE,), jnp.float32),
    ],
    mesh=_sc_mesh(),
    scratch_types=[
        pltpu.VMEM((CH_C,), jnp.int32),
        pltpu.VMEM((CH_C,), jnp.float32),
        pltpu.VMEM((RPT,), jnp.float32),
        pltpu.VMEM_SHARED((N_NODE,), jnp.float32),
    ],
)
def _sc_cnt(dst_ht, dst_th, zrows1, ones1,
            cnt_t, cnt_h,
            didx, ones_v, cbuf, acc_cnt):
    c = lax.axis_index("c")
    s = lax.axis_index("s")
    ebase = s * EPT

    pltpu.sync_copy(zrows1, cbuf)
    _zero_1d(cbuf, acc_cnt, s)
    pltpu.sync_copy(ones1, ones_v)
    plsc.subcore_barrier()

    def count_chunks(dst):
        def chunk(i, carry):
            base = ebase + i * CH_C
            pltpu.sync_copy(dst.at[pl.ds(base, CH_C)], didx)
            pltpu.sync_copy(ones_v, acc_cnt.at[didx], add=True)
            return carry

        lax.fori_loop(0, EPT // CH_C, chunk, 0)

    @pl.when(c == 0)
    def _():
        count_chunks(dst_ht)

    @pl.when(c == 1)
    def _():
        count_chunks(dst_th)

    plsc.subcore_barrier()

    @pl.when(c == 0)
    def _():
        _flush_1d(acc_cnt, cbuf, cnt_t, s)

    @pl.when(c == 1)
    def _():
        _flush_1d(acc_cnt, cbuf, cnt_h, s)


@functools.partial(
    pl.kernel,
    out_type=[
        jax.ShapeDtypeStruct((B, D), jnp.float32),
        jax.ShapeDtypeStruct((B, D), jnp.float32),
    ],
    mesh=_sc_mesh(),
    scratch_types=[
        pltpu.VMEM((BPT,), jnp.int32),
        pltpu.VMEM((DCH, D), jnp.float32),
    ],
)
def _sc_dec_gather(h_tail, eli0, eli1, z_src, z_dst, idx_v, rows):
    c = lax.axis_index("c")
    s = lax.axis_index("s")
    base = s * BPT

    def gather_to(eli, z_out):
        pltpu.sync_copy(eli.at[pl.ds(base, BPT)], idx_v)

        def chunk(k, carry):
            pltpu.sync_copy(h_tail.at[idx_v.at[pl.ds(k * DCH, DCH)]], rows)
            pltpu.sync_copy(rows, z_out.at[pl.ds(base + k * DCH, DCH)])
            return carry

        lax.fori_loop(0, BPT // DCH, chunk, 0)

    @pl.when(c == 0)
    def _():
        gather_to(eli0, z_src)

    @pl.when(c == 1)
    def _():
        gather_to(eli1, z_dst)


_TCR = 1000


def _tc_layer_body(xh, xt, StL, StR, ShL, ShR, cnt_t, cnt_h,
                   Wsh, Wnth, bh, Wst, Wnht, bt,
                   oh_L, oh_R, ot_L, ot_R):
    f32 = jnp.float32
    rt = 1.0 / jnp.maximum(cnt_t[...], 1.0)
    rh = 1.0 / jnp.maximum(cnt_h[...], 1.0)
    nh = jnp.dot(xh[...], Wsh[...], preferred_element_type=f32)
    nh = nh + jnp.dot(ShL[...] * rh, Wnth[:DH, :], preferred_element_type=f32)
    nh = nh + jnp.dot(ShR[...] * rh, Wnth[DH:, :], preferred_element_type=f32)
    nh = jnp.maximum(nh + bh[...], 0.0)
    oh_L[...] = nh[:, :DH]
    oh_R[...] = nh[:, DH:]
    nt = jnp.dot(xt[...], Wst[...], preferred_element_type=f32)
    nt = nt + jnp.dot(StL[...] * rt, Wnht[:DH, :], preferred_element_type=f32)
    nt = nt + jnp.dot(StR[...] * rt, Wnht[DH:, :], preferred_element_type=f32)
    nt = jnp.maximum(nt + bt[...], 0.0)
    ot_L[...] = nt[:, :DH]
    ot_R[...] = nt[:, DH:]


def _row_spec(w):
    return pl.BlockSpec((_TCR, w), lambda i: (i, 0))


def _full_spec(shape):
    return pl.BlockSpec(shape, lambda i: (0,) * len(shape))


def _tc_layer(xh, xt, StL, StR, ShL, ShR, cnt_t, cnt_h,
              Wsh, Wnth, bh, Wst, Wnht, bt):
    grid = (N_NODE // _TCR,)
    half = jax.ShapeDtypeStruct((N_NODE, DH), jnp.float32)
    return pl.pallas_call(
        _tc_layer_body,
        grid=grid,
        in_specs=[
            _row_spec(D), _row_spec(D),
            _row_spec(DH), _row_spec(DH), _row_spec(DH), _row_spec(DH),
            _row_spec(1), _row_spec(1),
            _full_spec((D, D)), _full_spec((D, D)), _full_spec((1, D)),
            _full_spec((D, D)), _full_spec((D, D)), _full_spec((1, D)),
        ],
        out_specs=[_row_spec(DH)] * 4,
        out_shape=[half] * 4,
        compiler_params=pltpu.CompilerParams(
            dimension_semantics=("parallel",)),
    )(xh, xt, StL, StR, ShL, ShR, cnt_t, cnt_h,
      Wsh, Wnth, bh, Wst, Wnht, bt)


def _tc_tail_body(htL, htR, StL, StR, cnt_t, Wst, Wnht, bt, out):
    f32 = jnp.float32
    rt = 1.0 / jnp.maximum(cnt_t[...], 1.0)
    nt = jnp.dot(htL[...], Wst[:DH, :], preferred_element_type=f32)
    nt = nt + jnp.dot(htR[...], Wst[DH:, :], preferred_element_type=f32)
    nt = nt + jnp.dot(StL[...] * rt, Wnht[:DH, :], preferred_element_type=f32)
    nt = nt + jnp.dot(StR[...] * rt, Wnht[DH:, :], preferred_element_type=f32)
    out[...] = jnp.maximum(nt + bt[...], 0.0)


def _tc_tail(htL, htR, StL, StR, cnt_t, Wst, Wnht, bt):
    grid = (N_NODE // _TCR,)
    return pl.pallas_call(
        _tc_tail_body,
        grid=grid,
        in_specs=[
            _row_spec(DH), _row_spec(DH), _row_spec(DH), _row_spec(DH),
            _row_spec(1),
            _full_spec((D, D)), _full_spec((D, D)), _full_spec((1, D)),
        ],
        out_specs=_row_spec(D),
        out_shape=jax.ShapeDtypeStruct((N_NODE, D), jnp.float32),
        compiler_params=pltpu.CompilerParams(
            dimension_semantics=("parallel",)),
    )(htL, htR, StL, StR, cnt_t, Wst, Wnht, bt)


def _tc_dec_body(zs, zd, W1a, W1b, b1, W2, b2, out):
    f32 = jnp.float32
    x = jnp.dot(zs[...], W1a[...], preferred_element_type=f32)
    x = x + jnp.dot(zd[...], W1b[...], preferred_element_type=f32)
    x = jnp.maximum(x + b1[...], 0.0)
    out[...] = jnp.dot(x, W2[...], preferred_element_type=f32) + b2[...]


def _tc_dec(z_src, z_dst, W1a, W1b, b1, W2, b2):
    return pl.pallas_call(
        _tc_dec_body,
        out_shape=jax.ShapeDtypeStruct((B, 1), jnp.float32),
    )(z_src, z_dst, W1a, W1b, b1, W2, b2)


def kernel(x_head, x_tail, edge_index_ht, edge_index_th, edge_label_index,
           Wsh0, Wnth0, bh0, Wst0, Wnht0, bt0,
           Wsh1, Wnth1, bh1, Wst1, Wnht1, bt1,
           lin1_W, lin1_b, lin2_W, lin2_b):
    i32 = jnp.int32
    src_ht = edge_index_ht[0].astype(i32)
    dst_ht = edge_index_ht[1].astype(i32)
    src_th = edge_index_th[0].astype(i32)
    dst_th = edge_index_th[1].astype(i32)
    eli0 = edge_label_index[0].astype(i32)
    eli1 = edge_label_index[1].astype(i32)

    xh_L, xh_R = x_head[:, :DH], x_head[:, DH:]
    xt_L, xt_R = x_tail[:, :DH], x_tail[:, DH:]

    zrows = jnp.zeros((RPT, DH), jnp.float32)
    zrows1 = jnp.zeros((RPT,), jnp.float32)
    ones1 = jnp.ones((CH_C,), jnp.float32)

    cnt_t1, cnt_h1 = _sc_cnt(dst_ht, dst_th, zrows1, ones1)
    cnt_t = cnt_t1.reshape(N_NODE, 1)
    cnt_h = cnt_h1.reshape(N_NODE, 1)
    StL, StR = _sc_agg(xh_L, xh_R, src_ht, dst_ht, zrows)
    ShL, ShR = _sc_agg(xt_L, xt_R, src_th, dst_th, zrows)

    h1hL, h1hR, h1tL, h1tR = _tc_layer(
        x_head, x_tail, StL, StR, ShL, ShR, cnt_t, cnt_h,
        Wsh0, Wnth0, bh0.reshape(1, D), Wst0, Wnht0, bt0.reshape(1, D))

    S1tL, S1tR = _sc_agg(h1hL, h1hR, src_ht, dst_ht, zrows)

    h2_tail = _tc_tail(h1tL, h1tR, S1tL, S1tR, cnt_t,
                       Wst1, Wnht1, bt1.reshape(1, D))

    z_src, z_dst = _sc_dec_gather(h2_tail, eli0, eli1)

    out = _tc_dec(z_src, z_dst, lin1_W[:D], lin1_W[D:],
                  lin1_b.reshape(1, D), lin2_W, lin2_b.reshape(1, 1))
    return out.reshape(-1)


# speedup vs baseline: 5.1068x; 1.0181x over previous
import functools

import jax
import jax.numpy as jnp
from jax import lax
from jax.experimental import pallas as pl
from jax.experimental.pallas import tpu as pltpu
from jax.experimental.pallas import tpu_sc as plsc

N_NODE = 10000
E = 160000
D = 256
DH = 128
B = 4096
NS = 16
EPT = E // NS
CH = 40
RPT = 624
REM_BASE = RPT * NS
REM = N_NODE - REM_BASE
BPT = B // NS
DCH = 8

_sc_mesh = functools.partial(
    plsc.VectorSubcoreMesh, core_axis_name="c", subcore_axis_name="s")


def _zero_acc(zrows, acc, s):
    pltpu.sync_copy(zrows.at[pl.ds(0, RPT)], acc.at[pl.ds(s * RPT, RPT)])

    @pl.when(s == NS - 1)
    def _():
        pltpu.sync_copy(zrows.at[pl.ds(0, REM)], acc.at[pl.ds(REM_BASE, REM)])


def _flush(acc, out, s):
    pltpu.sync_copy(acc.at[pl.ds(s * RPT, RPT)], out.at[pl.ds(s * RPT, RPT)])

    @pl.when(s == NS - 1)
    def _():
        pltpu.sync_copy(acc.at[pl.ds(REM_BASE, REM)],
                        out.at[pl.ds(REM_BASE, REM)])


NSLOT = 5
NCHUNK = EPT // CH
NGROUP = NCHUNK // NSLOT


def _agg_pipelined(tbl, src, dst, acc, ebase, slots,
                   cnt_flag=None, ones_v=None, acc_cnt=None):
    def idx_start(k, sl):
        base = ebase + k * CH
        pltpu.async_copy(src.at[pl.ds(base, CH)], sl["si"], sl["sem_i"])
        pltpu.async_copy(dst.at[pl.ds(base, CH)], sl["di"], sl["sem_i"])

    def idx_wait(sl):
        pltpu.make_async_copy(src.at[pl.ds(ebase, CH)], sl["si"],
                              sl["sem_i"]).wait()
        pltpu.make_async_copy(dst.at[pl.ds(ebase, CH)], sl["di"],
                              sl["sem_i"]).wait()

    def gather_start(sl):
        pltpu.async_copy(tbl.at[sl["si"]], sl["rows"], sl["sem_g"])

    def gather_wait(sl):
        pltpu.make_async_copy(tbl.at[sl["si"]], sl["rows"],
                              sl["sem_g"]).wait()

    def scat_start(sl):
        pltpu.async_copy(sl["rows"], acc.at[sl["di"]], sl["sem_s"], add=True)
        if cnt_flag is not None:
            @pl.when(cnt_flag)
            def _():
                pltpu.async_copy(ones_v, acc_cnt.at[sl["di"]], sl["sem_s"],
                                 add=True)

    def scat_drain(sl):
        pltpu.make_async_copy(sl["rows"], acc.at[sl["di"]],
                              sl["sem_s"]).wait()
        if cnt_flag is not None:
            @pl.when(cnt_flag)
            def _():
                pltpu.make_async_copy(ones_v, acc_cnt.at[sl["di"]],
                                      sl["sem_s"]).wait()

    idx_start(0, slots[0])
    idx_start(1, slots[1])

    def step(k, u):
        cur = slots[u]
        prev = slots[(u - 1) % NSLOT]
        prev2 = slots[(u - 2) % NSLOT]
        ahead2 = slots[(u + 2) % NSLOT]

        idx_wait(cur)
        gather_start(cur)

        @pl.when(k >= 1)
        def _():
            gather_wait(prev)
            scat_start(prev)

        @pl.when(k >= 2)
        def _():
            scat_drain(prev2)

        @pl.when(k + 2 < NCHUNK)
        def _():
            idx_start(k + 2, ahead2)

    def group(jj, carry):
        for u in range(NSLOT):
            step(jj * NSLOT + u, u)
        return carry

    lax.fori_loop(0, NGROUP, group, 0)

    last = slots[(NCHUNK - 1) % NSLOT]
    gather_wait(last)
    scat_start(last)
    scat_drain(slots[(NCHUNK - 2) % NSLOT])
    scat_drain(last)


def _zero_1d(zbuf, acc, s):
    pltpu.sync_copy(zbuf.at[pl.ds(0, RPT)], acc.at[pl.ds(s * RPT, RPT)])

    @pl.when(s == NS - 1)
    def _():
        pltpu.sync_copy(zbuf.at[pl.ds(0, REM)], acc.at[pl.ds(REM_BASE, REM)])


def _flush_1d(acc, cbuf, out, s):
    pltpu.sync_copy(acc.at[pl.ds(s * RPT, RPT)], cbuf.at[pl.ds(0, RPT)])
    pltpu.sync_copy(cbuf.at[pl.ds(0, RPT)], out.at[pl.ds(s * RPT, RPT)])

    @pl.when(s == NS - 1)
    def _():
        pltpu.sync_copy(acc.at[pl.ds(REM_BASE, REM)], cbuf.at[pl.ds(0, REM)])
        pltpu.sync_copy(cbuf.at[pl.ds(0, REM)], out.at[pl.ds(REM_BASE, REM)])


@functools.partial(
    pl.kernel,
    out_type=[
        jax.ShapeDtypeStruct((N_NODE, DH), jnp.float32),
        jax.ShapeDtypeStruct((N_NODE, DH), jnp.float32),
        jax.ShapeDtypeStruct((N_NODE, DH), jnp.float32),
        jax.ShapeDtypeStruct((N_NODE, DH), jnp.float32),
        jax.ShapeDtypeStruct((N_NODE,), jnp.float32),
        jax.ShapeDtypeStruct((N_NODE,), jnp.float32),
    ],
    mesh=_sc_mesh(),
    scratch_types=(
        [pltpu.VMEM((CH,), jnp.int32)] * (2 * NSLOT)
        + [pltpu.VMEM((CH, DH), jnp.float32)] * NSLOT
        + [pltpu.VMEM_SHARED((N_NODE, DH), jnp.float32)]
        + [pltpu.SemaphoreType.DMA] * (3 * NSLOT)
        + [pltpu.VMEM_SHARED((N_NODE,), jnp.float32),
           pltpu.VMEM((CH,), jnp.float32),
           pltpu.VMEM((RPT,), jnp.float32),
           pltpu.VMEM((16,), jnp.int32)]
    ),
)
def _sc_agg(h1L, h1R, src1, dst1, h2L, h2R, src2, dst2,
            zrows, zc, ones1, flagv,
            S1L, S1R, S2L, S2R, cnt1, cnt2, *scr):
    sis = scr[0:NSLOT]
    dis = scr[NSLOT:2 * NSLOT]
    rows = scr[2 * NSLOT:3 * NSLOT]
    acc = scr[3 * NSLOT]
    sem_i = scr[3 * NSLOT + 1:3 * NSLOT + 1 + NSLOT]
    sem_g = scr[3 * NSLOT + 1 + NSLOT:3 * NSLOT + 1 + 2 * NSLOT]
    sem_s = scr[3 * NSLOT + 1 + 2 * NSLOT:3 * NSLOT + 1 + 3 * NSLOT]
    acc_cnt, ones_v, cbuf, flag_v = scr[3 * NSLOT + 1 + 3 * NSLOT:]
    slots = [dict(si=sis[u], di=dis[u], rows=rows[u],
                  sem_i=sem_i[u], sem_g=sem_g[u], sem_s=sem_s[u])
             for u in range(NSLOT)]

    c = lax.axis_index("c")
    s = lax.axis_index("s")
    ebase = s * EPT

    pltpu.sync_copy(flagv, flag_v)
    both = flag_v[...][0] > 0

    _zero_acc(zrows, acc, s)
    pltpu.sync_copy(zc, cbuf)
    _zero_1d(cbuf, acc_cnt, s)
    pltpu.sync_copy(ones1, ones_v)
    plsc.subcore_barrier()

    @pl.when(c == 0)
    def _():
        _agg_pipelined(h1L, src1, dst1, acc, ebase, slots,
                       cnt_flag=both, ones_v=ones_v, acc_cnt=acc_cnt)

    @pl.when(c == 1)
    def _():
        _agg_pipelined(h1R, src1, dst1, acc, ebase, slots)

    plsc.subcore_barrier()

    @pl.when(c == 0)
    def _():
        _flush(acc, S1L, s)

        @pl.when(both)
        def _():
            _flush_1d(acc_cnt, cbuf, cnt1, s)

    @pl.when(c == 1)
    def _():
        _flush(acc, S1R, s)

    plsc.subcore_barrier()

    @pl.when(both)
    def _():
        _zero_acc(zrows, acc, s)
        plsc.subcore_barrier()

        @pl.when(c == 0)
        def _():
            _agg_pipelined(h2L, src2, dst2, acc, ebase, slots)

        @pl.when(c == 1)
        def _():
            _agg_pipelined(h2R, src2, dst2, acc, ebase, slots,
                           cnt_flag=both, ones_v=ones_v, acc_cnt=acc_cnt)

        plsc.subcore_barrier()

        @pl.when(c == 0)
        def _():
            _flush(acc, S2L, s)

        @pl.when(c == 1)
        def _():
            _flush(acc, S2R, s)
            _flush_1d(acc_cnt, cbuf, cnt2, s)


@functools.partial(
    pl.kernel,
    out_type=[
        jax.ShapeDtypeStruct((B, D), jnp.float32),
        jax.ShapeDtypeStruct((B, D), jnp.float32),
    ],
    mesh=_sc_mesh(),
    scratch_types=(
        [pltpu.VMEM((BPT,), jnp.int32)]
        + [pltpu.VMEM((DCH, D), jnp.float32)] * 4
        + [pltpu.SemaphoreType.DMA] * 8
    ),
)
def _sc_dec_gather(h_tail, eli0, eli1, z_src, z_dst, idx_v, *scr):
    rows = scr[0:4]
    sem_g = scr[4:8]
    sem_w = scr[8:12]
    c = lax.axis_index("c")
    s = lax.axis_index("s")
    base = s * BPT
    nch = BPT // DCH

    def gather_to(eli, z_out):
        pltpu.sync_copy(eli.at[pl.ds(base, BPT)], idx_v)

        def g_start(k, u):
            pltpu.async_copy(h_tail.at[idx_v.at[pl.ds(k * DCH, DCH)]],
                             rows[u], sem_g[u])

        def g_wait(u):
            pltpu.make_async_copy(h_tail.at[idx_v.at[pl.ds(0, DCH)]],
                                  rows[u], sem_g[u]).wait()

        def w_start(k, u):
            pltpu.async_copy(rows[u], z_out.at[pl.ds(base + k * DCH, DCH)],
                             sem_w[u])

        def w_drain(u):
            pltpu.make_async_copy(rows[u], z_out.at[pl.ds(base, DCH)],
                                  sem_w[u]).wait()

        def step(k, u):
            g_start(k, u)

            @pl.when(k >= 1)
            def _():
                g_wait((u - 1) % 4)
                w_start(k - 1, (u - 1) % 4)

            @pl.when(k >= 2)
            def _():
                w_drain((u - 2) % 4)

        def group(jj, carry):
            for u in range(4):
                step(jj * 4 + u, u)
            return carry

        lax.fori_loop(0, nch // 4, group, 0)
        g_wait((nch - 1) % 4)
        w_start(nch - 1, (nch - 1) % 4)
        w_drain((nch - 2) % 4)
        w_drain((nch - 1) % 4)

    @pl.when(c == 0)
    def _():
        gather_to(eli0, z_src)

    @pl.when(c == 1)
    def _():
        gather_to(eli1, z_dst)


_TCR = 1000


def _tc_layer_body(xh, xt, StL, StR, ShL, ShR, cnt_t, cnt_h,
                   Wsh, Wnth, bh, Wst, Wnht, bt,
                   oh_L, oh_R, ot_L, ot_R):
    f32 = jnp.float32
    rt = 1.0 / jnp.maximum(cnt_t[...], 1.0)
    rh = 1.0 / jnp.maximum(cnt_h[...], 1.0)
    nh = jnp.dot(xh[...], Wsh[...], preferred_element_type=f32)
    nh = nh + jnp.dot(ShL[...] * rh, Wnth[:DH, :], preferred_element_type=f32)
    nh = nh + jnp.dot(ShR[...] * rh, Wnth[DH:, :], preferred_element_type=f32)
    nh = jnp.maximum(nh + bh[...], 0.0)
    oh_L[...] = nh[:, :DH]
    oh_R[...] = nh[:, DH:]
    nt = jnp.dot(xt[...], Wst[...], preferred_element_type=f32)
    nt = nt + jnp.dot(StL[...] * rt, Wnht[:DH, :], preferred_element_type=f32)
    nt = nt + jnp.dot(StR[...] * rt, Wnht[DH:, :], preferred_element_type=f32)
    nt = jnp.maximum(nt + bt[...], 0.0)
    ot_L[...] = nt[:, :DH]
    ot_R[...] = nt[:, DH:]


def _row_spec(w):
    return pl.BlockSpec((_TCR, w), lambda i: (i, 0))


def _full_spec(shape):
    return pl.BlockSpec(shape, lambda i: (0,) * len(shape))


def _tc_layer(xh, xt, StL, StR, ShL, ShR, cnt_t, cnt_h,
              Wsh, Wnth, bh, Wst, Wnht, bt):
    grid = (N_NODE // _TCR,)
    half = jax.ShapeDtypeStruct((N_NODE, DH), jnp.float32)
    return pl.pallas_call(
        _tc_layer_body,
        grid=grid,
        in_specs=[
            _row_spec(D), _row_spec(D),
            _row_spec(DH), _row_spec(DH), _row_spec(DH), _row_spec(DH),
            _row_spec(1), _row_spec(1),
            _full_spec((D, D)), _full_spec((D, D)), _full_spec((1, D)),
            _full_spec((D, D)), _full_spec((D, D)), _full_spec((1, D)),
        ],
        out_specs=[_row_spec(DH)] * 4,
        out_shape=[half] * 4,
        compiler_params=pltpu.CompilerParams(
            dimension_semantics=("parallel",)),
    )(xh, xt, StL, StR, ShL, ShR, cnt_t, cnt_h,
      Wsh, Wnth, bh, Wst, Wnht, bt)


def _tc_tail_body(htL, htR, StL, StR, cnt_t, Wst, Wnht, bt, out):
    f32 = jnp.float32
    rt = 1.0 / jnp.maximum(cnt_t[...], 1.0)
    nt = jnp.dot(htL[...], Wst[:DH, :], preferred_element_type=f32)
    nt = nt + jnp.dot(htR[...], Wst[DH:, :], preferred_element_type=f32)
    nt = nt + jnp.dot(StL[...] * rt, Wnht[:DH, :], preferred_element_type=f32)
    nt = nt + jnp.dot(StR[...] * rt, Wnht[DH:, :], preferred_element_type=f32)
    out[...] = jnp.maximum(nt + bt[...], 0.0)


def _tc_tail(htL, htR, StL, StR, cnt_t, Wst, Wnht, bt):
    grid = (N_NODE // _TCR,)
    return pl.pallas_call(
        _tc_tail_body,
        grid=grid,
        in_specs=[
            _row_spec(DH), _row_spec(DH), _row_spec(DH), _row_spec(DH),
            _row_spec(1),
            _full_spec((D, D)), _full_spec((D, D)), _full_spec((1, D)),
        ],
        out_specs=_row_spec(D),
        out_shape=jax.ShapeDtypeStruct((N_NODE, D), jnp.float32),
        compiler_params=pltpu.CompilerParams(
            dimension_semantics=("parallel",)),
    )(htL, htR, StL, StR, cnt_t, Wst, Wnht, bt)


def _tc_dec_body(zs, zd, W1a, W1b, b1, W2, b2, out):
    f32 = jnp.float32
    x = jnp.dot(zs[...], W1a[...], preferred_element_type=f32)
    x = x + jnp.dot(zd[...], W1b[...], preferred_element_type=f32)
    x = jnp.maximum(x + b1[...], 0.0)
    out[...] = jnp.dot(x, W2[...], preferred_element_type=f32) + b2[...]


def _tc_dec(z_src, z_dst, W1a, W1b, b1, W2, b2):
    return pl.pallas_call(
        _tc_dec_body,
        out_shape=jax.ShapeDtypeStruct((B, 1), jnp.float32),
    )(z_src, z_dst, W1a, W1b, b1, W2, b2)


def kernel(x_head, x_tail, edge_index_ht, edge_index_th, edge_label_index,
           Wsh0, Wnth0, bh0, Wst0, Wnht0, bt0,
           Wsh1, Wnth1, bh1, Wst1, Wnht1, bt1,
           lin1_W, lin1_b, lin2_W, lin2_b):
    i32 = jnp.int32
    src_ht = edge_index_ht[0].astype(i32)
    dst_ht = edge_index_ht[1].astype(i32)
    src_th = edge_index_th[0].astype(i32)
    dst_th = edge_index_th[1].astype(i32)
    eli0 = edge_label_index[0].astype(i32)
    eli1 = edge_label_index[1].astype(i32)

    xh_L, xh_R = x_head[:, :DH], x_head[:, DH:]
    xt_L, xt_R = x_tail[:, :DH], x_tail[:, DH:]

    zrows = jnp.zeros((RPT, DH), jnp.float32)
    zrows1 = jnp.zeros((RPT,), jnp.float32)
    ones1 = jnp.ones((CH,), jnp.float32)
    flag_on = jnp.ones((16,), jnp.int32)
    flag_off = jnp.zeros((16,), jnp.int32)

    StL, StR, ShL, ShR, cnt_t1, cnt_h1 = _sc_agg(
        xh_L, xh_R, src_ht, dst_ht, xt_L, xt_R, src_th, dst_th,
        zrows, zrows1, ones1, flag_on)
    cnt_t = cnt_t1.reshape(N_NODE, 1)
    cnt_h = cnt_h1.reshape(N_NODE, 1)

    h1hL, h1hR, h1tL, h1tR = _tc_layer(
        x_head, x_tail, StL, StR, ShL, ShR, cnt_t, cnt_h,
        Wsh0, Wnth0, bh0.reshape(1, D), Wst0, Wnht0, bt0.reshape(1, D))

    S1tL, S1tR, _u1, _u2, _u3, _u4 = _sc_agg(
        h1hL, h1hR, src_ht, dst_ht, h1tL, h1tR, src_th, dst_th,
        zrows, zrows1, ones1, flag_off)

    h2_tail = _tc_tail(h1tL, h1tR, S1tL, S1tR, cnt_t,
                       Wst1, Wnht1, bt1.reshape(1, D))

    z_src, z_dst = _sc_dec_gather(h2_tail, eli0, eli1)

    out = _tc_dec(z_src, z_dst, lin1_W[:D], lin1_W[D:],
                  lin1_b.reshape(1, D), lin2_W, lin2_b.reshape(1, 1))
    return out.reshape(-1)


# speedup vs baseline: 5.1410x; 1.0067x over previous
import functools

import jax
import jax.numpy as jnp
from jax import lax
from jax.experimental import pallas as pl
from jax.experimental.pallas import tpu as pltpu
from jax.experimental.pallas import tpu_sc as plsc

N_NODE = 10000
E = 160000
D = 256
DH = 128
B = 4096
NS = 16
EPT = E // NS
CH = 40
RPT = 624
REM_BASE = RPT * NS
REM = N_NODE - REM_BASE
BPT = B // NS
DCH = 8

_sc_mesh = functools.partial(
    plsc.VectorSubcoreMesh, core_axis_name="c", subcore_axis_name="s")


def _zero_acc(zrows, acc, s):
    pltpu.sync_copy(zrows.at[pl.ds(0, RPT)], acc.at[pl.ds(s * RPT, RPT)])

    @pl.when(s == NS - 1)
    def _():
        pltpu.sync_copy(zrows.at[pl.ds(0, REM)], acc.at[pl.ds(REM_BASE, REM)])


def _flush(acc, out, s):
    pltpu.sync_copy(acc.at[pl.ds(s * RPT, RPT)], out.at[pl.ds(s * RPT, RPT)])

    @pl.when(s == NS - 1)
    def _():
        pltpu.sync_copy(acc.at[pl.ds(REM_BASE, REM)],
                        out.at[pl.ds(REM_BASE, REM)])


NSLOT = 5
NCHUNK = EPT // CH
NGROUP = NCHUNK // NSLOT


def _agg_pipelined(tbl, src, dst, acc, ebase, slots, sblks,
                   cnt_flag=None, ones_v=None, acc_cnt=None):
    GB = NSLOT * CH

    def blk_start(g, sb):
        pltpu.async_copy(src.at[pl.ds(ebase + g * GB, GB)], sb["buf"],
                         sb["sem"])

    def blk_wait(sb):
        pltpu.make_async_copy(src.at[pl.ds(ebase, GB)], sb["buf"],
                              sb["sem"]).wait()

    def idx_start(k, sl):
        base = ebase + k * CH
        pltpu.async_copy(dst.at[pl.ds(base, CH)], sl["di"], sl["sem_i"])

    def idx_wait(sl):
        pltpu.make_async_copy(dst.at[pl.ds(ebase, CH)], sl["di"],
                              sl["sem_i"]).wait()

    def gather_start(sl, sb, u):
        pltpu.async_copy(tbl.at[sb["buf"].at[pl.ds(u * CH, CH)]],
                         sl["rows"], sl["sem_g"])

    def gather_wait(sl, sb, u):
        pltpu.make_async_copy(tbl.at[sb["buf"].at[pl.ds(u * CH, CH)]],
                              sl["rows"], sl["sem_g"]).wait()

    def scat_start(sl):
        pltpu.async_copy(sl["rows"], acc.at[sl["di"]], sl["sem_s"], add=True)
        if cnt_flag is not None:
            @pl.when(cnt_flag)
            def _():
                pltpu.async_copy(ones_v, acc_cnt.at[sl["di"]], sl["sem_s"],
                                 add=True)

    def scat_drain(sl):
        pltpu.make_async_copy(sl["rows"], acc.at[sl["di"]],
                              sl["sem_s"]).wait()
        if cnt_flag is not None:
            @pl.when(cnt_flag)
            def _():
                pltpu.make_async_copy(ones_v, acc_cnt.at[sl["di"]],
                                      sl["sem_s"]).wait()

    blk_start(0, sblks[0])
    blk_start(1, sblks[1])
    idx_start(0, slots[0])
    idx_start(1, slots[1])

    def step(k, u, sb, sb_other, refetch):
        cur = slots[u]
        prev = slots[(u - 1) % NSLOT]
        prev2 = slots[(u - 2) % NSLOT]
        ahead2 = slots[(u + 2) % NSLOT]

        idx_wait(cur)
        gather_start(cur, sb, u)

        @pl.when(k >= 1)
        def _():
            gather_wait(prev, sb if u != 0 else sb_other, (u - 1) % NSLOT)
            scat_start(prev)

        if u == 0:
            @pl.when((k >= 1) & (refetch < NGROUP))
            def _():
                blk_start(refetch, sb_other)

        @pl.when(k >= 2)
        def _():
            scat_drain(prev2)

        @pl.when(k + 2 < NCHUNK)
        def _():
            idx_start(k + 2, ahead2)

    def grouppair(p, carry):
        ga = 2 * p
        gb = 2 * p + 1
        blk_wait(sblks[0])
        for u in range(NSLOT):
            step(ga * NSLOT + u, u, sblks[0], sblks[1], ga + 1)
        blk_wait(sblks[1])
        for u in range(NSLOT):
            step(gb * NSLOT + u, u, sblks[1], sblks[0], gb + 1)
        return carry

    lax.fori_loop(0, NGROUP // 2, grouppair, 0)

    last = slots[(NCHUNK - 1) % NSLOT]
    gather_wait(last, sblks[1], (NCHUNK - 1) % NSLOT)
    scat_start(last)
    scat_drain(slots[(NCHUNK - 2) % NSLOT])
    scat_drain(last)


def _zero_1d(zbuf, acc, s):
    pltpu.sync_copy(zbuf.at[pl.ds(0, RPT)], acc.at[pl.ds(s * RPT, RPT)])

    @pl.when(s == NS - 1)
    def _():
        pltpu.sync_copy(zbuf.at[pl.ds(0, REM)], acc.at[pl.ds(REM_BASE, REM)])


def _flush_1d(acc, cbuf, out, s):
    pltpu.sync_copy(acc.at[pl.ds(s * RPT, RPT)], cbuf.at[pl.ds(0, RPT)])
    pltpu.sync_copy(cbuf.at[pl.ds(0, RPT)], out.at[pl.ds(s * RPT, RPT)])

    @pl.when(s == NS - 1)
    def _():
        pltpu.sync_copy(acc.at[pl.ds(REM_BASE, REM)], cbuf.at[pl.ds(0, REM)])
        pltpu.sync_copy(cbuf.at[pl.ds(0, REM)], out.at[pl.ds(REM_BASE, REM)])


@functools.partial(
    pl.kernel,
    out_type=[
        jax.ShapeDtypeStruct((N_NODE, DH), jnp.float32),
        jax.ShapeDtypeStruct((N_NODE, DH), jnp.float32),
        jax.ShapeDtypeStruct((N_NODE, DH), jnp.float32),
        jax.ShapeDtypeStruct((N_NODE, DH), jnp.float32),
        jax.ShapeDtypeStruct((N_NODE,), jnp.float32),
        jax.ShapeDtypeStruct((N_NODE,), jnp.float32),
    ],
    mesh=_sc_mesh(),
    scratch_types=(
        [pltpu.VMEM((CH,), jnp.int32)] * NSLOT
        + [pltpu.VMEM((NSLOT * CH,), jnp.int32)] * 2
        + [pltpu.VMEM((CH, DH), jnp.float32)] * NSLOT
        + [pltpu.VMEM_SHARED((N_NODE, DH), jnp.float32)]
        + [pltpu.SemaphoreType.DMA] * (3 * NSLOT + 2)
        + [pltpu.VMEM_SHARED((N_NODE,), jnp.float32),
           pltpu.VMEM((CH,), jnp.float32),
           pltpu.VMEM((RPT,), jnp.float32),
           pltpu.VMEM((16,), jnp.int32)]
    ),
)
def _sc_agg(h1L, h1R, src1, dst1, h2L, h2R, src2, dst2,
            zrows, zc, ones1, flagv,
            S1L, S1R, S2L, S2R, cnt1, cnt2, *scr):
    dis = scr[0:NSLOT]
    blkbufs = scr[NSLOT:NSLOT + 2]
    rows = scr[NSLOT + 2:2 * NSLOT + 2]
    acc = scr[2 * NSLOT + 2]
    b0 = 2 * NSLOT + 3
    sem_i = scr[b0:b0 + NSLOT]
    sem_g = scr[b0 + NSLOT:b0 + 2 * NSLOT]
    sem_s = scr[b0 + 2 * NSLOT:b0 + 3 * NSLOT]
    sem_b = scr[b0 + 3 * NSLOT:b0 + 3 * NSLOT + 2]
    acc_cnt, ones_v, cbuf, flag_v = scr[b0 + 3 * NSLOT + 2:]
    slots = [dict(di=dis[u], rows=rows[u],
                  sem_i=sem_i[u], sem_g=sem_g[u], sem_s=sem_s[u])
             for u in range(NSLOT)]
    sblks = [dict(buf=blkbufs[v], sem=sem_b[v]) for v in range(2)]

    c = lax.axis_index("c")
    s = lax.axis_index("s")
    ebase = s * EPT

    pltpu.sync_copy(flagv, flag_v)
    both = flag_v[...][0] > 0

    _zero_acc(zrows, acc, s)
    pltpu.sync_copy(zc, cbuf)
    _zero_1d(cbuf, acc_cnt, s)
    pltpu.sync_copy(ones1, ones_v)
    plsc.subcore_barrier()

    @pl.when(c == 0)
    def _():
        _agg_pipelined(h1L, src1, dst1, acc, ebase, slots, sblks,
                       cnt_flag=both, ones_v=ones_v, acc_cnt=acc_cnt)

    @pl.when(c == 1)
    def _():
        _agg_pipelined(h1R, src1, dst1, acc, ebase, slots, sblks)

    plsc.subcore_barrier()

    @pl.when(c == 0)
    def _():
        _flush(acc, S1L, s)

        @pl.when(both)
        def _():
            _flush_1d(acc_cnt, cbuf, cnt1, s)

    @pl.when(c == 1)
    def _():
        _flush(acc, S1R, s)

    plsc.subcore_barrier()

    @pl.when(both)
    def _():
        _zero_acc(zrows, acc, s)
        plsc.subcore_barrier()

        @pl.when(c == 0)
        def _():
            _agg_pipelined(h2L, src2, dst2, acc, ebase, slots, sblks)

        @pl.when(c == 1)
        def _():
            _agg_pipelined(h2R, src2, dst2, acc, ebase, slots, sblks,
                           cnt_flag=both, ones_v=ones_v, acc_cnt=acc_cnt)

        plsc.subcore_barrier()

        @pl.when(c == 0)
        def _():
            _flush(acc, S2L, s)

        @pl.when(c == 1)
        def _():
            _flush(acc, S2R, s)
            _flush_1d(acc_cnt, cbuf, cnt2, s)


@functools.partial(
    pl.kernel,
    out_type=[
        jax.ShapeDtypeStruct((B, D), jnp.float32),
        jax.ShapeDtypeStruct((B, D), jnp.float32),
    ],
    mesh=_sc_mesh(),
    scratch_types=(
        [pltpu.VMEM((BPT,), jnp.int32)]
        + [pltpu.VMEM((DCH, D), jnp.float32)] * 4
        + [pltpu.SemaphoreType.DMA] * 8
    ),
)
def _sc_dec_gather(h_tail, eli0, eli1, z_src, z_dst, idx_v, *scr):
    rows = scr[0:4]
    sem_g = scr[4:8]
    sem_w = scr[8:12]
    c = lax.axis_index("c")
    s = lax.axis_index("s")
    base = s * BPT
    nch = BPT // DCH

    def gather_to(eli, z_out):
        pltpu.sync_copy(eli.at[pl.ds(base, BPT)], idx_v)

        def g_start(k, u):
            pltpu.async_copy(h_tail.at[idx_v.at[pl.ds(k * DCH, DCH)]],
                             rows[u], sem_g[u])

        def g_wait(u):
            pltpu.make_async_copy(h_tail.at[idx_v.at[pl.ds(0, DCH)]],
                                  rows[u], sem_g[u]).wait()

        def w_start(k, u):
            pltpu.async_copy(rows[u], z_out.at[pl.ds(base + k * DCH, DCH)],
                             sem_w[u])

        def w_drain(u):
            pltpu.make_async_copy(rows[u], z_out.at[pl.ds(base, DCH)],
                                  sem_w[u]).wait()

        def step(k, u):
            g_start(k, u)

            @pl.when(k >= 1)
            def _():
                g_wait((u - 1) % 4)
                w_start(k - 1, (u - 1) % 4)

            @pl.when(k >= 2)
            def _():
                w_drain((u - 2) % 4)

        def group(jj, carry):
            for u in range(4):
                step(jj * 4 + u, u)
            return carry

        lax.fori_loop(0, nch // 4, group, 0)
        g_wait((nch - 1) % 4)
        w_start(nch - 1, (nch - 1) % 4)
        w_drain((nch - 2) % 4)
        w_drain((nch - 1) % 4)

    @pl.when(c == 0)
    def _():
        gather_to(eli0, z_src)

    @pl.when(c == 1)
    def _():
        gather_to(eli1, z_dst)


_TCR = 1000


def _tc_layer_body(xh, xt, StL, StR, ShL, ShR, cnt_t, cnt_h,
                   Wsh, Wnth, bh, Wst, Wnht, bt,
                   oh_L, oh_R, ot_L, ot_R):
    f32 = jnp.float32
    rt = 1.0 / jnp.maximum(cnt_t[...], 1.0)
    rh = 1.0 / jnp.maximum(cnt_h[...], 1.0)
    nh = jnp.dot(xh[...], Wsh[...], preferred_element_type=f32)
    nh = nh + jnp.dot(ShL[...] * rh, Wnth[:DH, :], preferred_element_type=f32)
    nh = nh + jnp.dot(ShR[...] * rh, Wnth[DH:, :], preferred_element_type=f32)
    nh = jnp.maximum(nh + bh[...], 0.0)
    oh_L[...] = nh[:, :DH]
    oh_R[...] = nh[:, DH:]
    nt = jnp.dot(xt[...], Wst[...], preferred_element_type=f32)
    nt = nt + jnp.dot(StL[...] * rt, Wnht[:DH, :], preferred_element_type=f32)
    nt = nt + jnp.dot(StR[...] * rt, Wnht[DH:, :], preferred_element_type=f32)
    nt = jnp.maximum(nt + bt[...], 0.0)
    ot_L[...] = nt[:, :DH]
    ot_R[...] = nt[:, DH:]


def _row_spec(w):
    return pl.BlockSpec((_TCR, w), lambda i: (i, 0))


def _full_spec(shape):
    return pl.BlockSpec(shape, lambda i: (0,) * len(shape))


def _tc_layer(xh, xt, StL, StR, ShL, ShR, cnt_t, cnt_h,
              Wsh, Wnth, bh, Wst, Wnht, bt):
    grid = (N_NODE // _TCR,)
    half = jax.ShapeDtypeStruct((N_NODE, DH), jnp.float32)
    return pl.pallas_call(
        _tc_layer_body,
        grid=grid,
        in_specs=[
            _row_spec(D), _row_spec(D),
            _row_spec(DH), _row_spec(DH), _row_spec(DH), _row_spec(DH),
            _row_spec(1), _row_spec(1),
            _full_spec((D, D)), _full_spec((D, D)), _full_spec((1, D)),
            _full_spec((D, D)), _full_spec((D, D)), _full_spec((1, D)),
        ],
        out_specs=[_row_spec(DH)] * 4,
        out_shape=[half] * 4,
        compiler_params=pltpu.CompilerParams(
            dimension_semantics=("parallel",)),
    )(xh, xt, StL, StR, ShL, ShR, cnt_t, cnt_h,
      Wsh, Wnth, bh, Wst, Wnht, bt)


def _tc_tail_body(htL, htR, StL, StR, cnt_t, Wst, Wnht, bt, out):
    f32 = jnp.float32
    rt = 1.0 / jnp.maximum(cnt_t[...], 1.0)
    nt = jnp.dot(htL[...], Wst[:DH, :], preferred_element_type=f32)
    nt = nt + jnp.dot(htR[...], Wst[DH:, :], preferred_element_type=f32)
    nt = nt + jnp.dot(StL[...] * rt, Wnht[:DH, :], preferred_element_type=f32)
    nt = nt + jnp.dot(StR[...] * rt, Wnht[DH:, :], preferred_element_type=f32)
    out[...] = jnp.maximum(nt + bt[...], 0.0)


def _tc_tail(htL, htR, StL, StR, cnt_t, Wst, Wnht, bt):
    grid = (N_NODE // _TCR,)
    return pl.pallas_call(
        _tc_tail_body,
        grid=grid,
        in_specs=[
            _row_spec(DH), _row_spec(DH), _row_spec(DH), _row_spec(DH),
            _row_spec(1),
            _full_spec((D, D)), _full_spec((D, D)), _full_spec((1, D)),
        ],
        out_specs=_row_spec(D),
        out_shape=jax.ShapeDtypeStruct((N_NODE, D), jnp.float32),
        compiler_params=pltpu.CompilerParams(
            dimension_semantics=("parallel",)),
    )(htL, htR, StL, StR, cnt_t, Wst, Wnht, bt)


def _tc_dec_body(zs, zd, W1a, W1b, b1, W2, b2, out):
    f32 = jnp.float32
    x = jnp.dot(zs[...], W1a[...], preferred_element_type=f32)
    x = x + jnp.dot(zd[...], W1b[...], preferred_element_type=f32)
    x = jnp.maximum(x + b1[...], 0.0)
    out[...] = jnp.dot(x, W2[...], preferred_element_type=f32) + b2[...]


def _tc_dec(z_src, z_dst, W1a, W1b, b1, W2, b2):
    return pl.pallas_call(
        _tc_dec_body,
        out_shape=jax.ShapeDtypeStruct((B, 1), jnp.float32),
    )(z_src, z_dst, W1a, W1b, b1, W2, b2)


def kernel(x_head, x_tail, edge_index_ht, edge_index_th, edge_label_index,
           Wsh0, Wnth0, bh0, Wst0, Wnht0, bt0,
           Wsh1, Wnth1, bh1, Wst1, Wnht1, bt1,
           lin1_W, lin1_b, lin2_W, lin2_b):
    i32 = jnp.int32
    src_ht = edge_index_ht[0].astype(i32)
    dst_ht = edge_index_ht[1].astype(i32)
    src_th = edge_index_th[0].astype(i32)
    dst_th = edge_index_th[1].astype(i32)
    eli0 = edge_label_index[0].astype(i32)
    eli1 = edge_label_index[1].astype(i32)

    xh_L, xh_R = x_head[:, :DH], x_head[:, DH:]
    xt_L, xt_R = x_tail[:, :DH], x_tail[:, DH:]

    zrows = jnp.zeros((RPT, DH), jnp.float32)
    zrows1 = jnp.zeros((RPT,), jnp.float32)
    ones1 = jnp.ones((CH,), jnp.float32)
    flag_on = jnp.ones((16,), jnp.int32)
    flag_off = jnp.zeros((16,), jnp.int32)

    StL, StR, ShL, ShR, cnt_t1, cnt_h1 = _sc_agg(
        xh_L, xh_R, src_ht, dst_ht, xt_L, xt_R, src_th, dst_th,
        zrows, zrows1, ones1, flag_on)
    cnt_t = cnt_t1.reshape(N_NODE, 1)
    cnt_h = cnt_h1.reshape(N_NODE, 1)

    h1hL, h1hR, h1tL, h1tR = _tc_layer(
        x_head, x_tail, StL, StR, ShL, ShR, cnt_t, cnt_h,
        Wsh0, Wnth0, bh0.reshape(1, D), Wst0, Wnht0, bt0.reshape(1, D))

    S1tL, S1tR, _u1, _u2, _u3, _u4 = _sc_agg(
        h1hL, h1hR, src_ht, dst_ht, h1tL, h1tR, src_th, dst_th,
        zrows, zrows1, ones1, flag_off)

    h2_tail = _tc_tail(h1tL, h1tR, S1tL, S1tR, cnt_t,
                       Wst1, Wnht1, bt1.reshape(1, D))

    z_src, z_dst = _sc_dec_gather(h2_tail, eli0, eli1)

    out = _tc_dec(z_src, z_dst, lin1_W[:D], lin1_W[D:],
                  lin1_b.reshape(1, D), lin2_W, lin2_b.reshape(1, 1))
    return out.reshape(-1)
